# baseline (speedup 1.0000x reference)
import functools

import jax
import jax.numpy as jnp
from jax import lax
from jax.experimental import pallas as pl
from jax.experimental.pallas import tpu as pltpu
from jax.experimental.pallas import tpu_sc as plsc

_VOCAB = 1000000
_HIDDEN = 64
_SEQ = 2048
_BATCH = 4
_NC = 2
_NS = 16
_NW = _NC * _NS
_B = _SEQ * _BATCH
_BPW = _B // _NW
_L = 16
_NG = _BPW // _L
_K = 4
_NB = _BPW // _K
_TW = 128


@functools.lru_cache(maxsize=1)
def _make_gather():
    mesh = plsc.VectorSubcoreMesh(core_axis_name="c", subcore_axis_name="s")

    @functools.partial(
        pl.kernel,
        mesh=mesh,
        out_type=jax.ShapeDtypeStruct((_BATCH, _HIDDEN, _SEQ), jnp.float32),
        scratch_types=[
            pltpu.VMEM((_BPW,), jnp.int32),
            pltpu.SMEM((_BPW,), jnp.int32),
            pltpu.VMEM((3 * _K, _HIDDEN, _TW), jnp.float32),
            pltpu.VMEM((_HIDDEN, _BPW), jnp.float32),
            pltpu.SemaphoreType.DMA,
            pltpu.SemaphoreType.DMA,
            pltpu.SemaphoreType.DMA,
        ],
        compiler_params=pltpu.CompilerParams(needs_layout_passes=False),
    )
    def gather_kernel(idx_hbm, table_hbm, out_hbm,
                      idx_v, id_s, slab_v, out_v, sem_a, sem_b, sem_c):
        wid = lax.axis_index("s") * _NC + lax.axis_index("c")
        b = wid >> 3
        l0 = (wid & 7) * _BPW
        pltpu.sync_copy(idx_hbm.at[wid], idx_v)

        lane = lax.iota(jnp.int32, _L)

        def spill_ids(g):
            vvec = idx_v[pl.ds(g * _L, _L)]
            for j in range(_L):
                id_s[g * _L + j] = jnp.max(jnp.where(lane == j, vvec, 0))

        hvecs = [hg * _L + lane for hg in range(_HIDDEN // _L)]
        sems = (sem_a, sem_b, sem_c)

        def fire(bt, region):
            base = bt * _K
            for j in range(_K):
                v = id_s[base + j]
                s = pl.multiple_of((v >> 7) * _TW, _TW)
                pltpu.async_copy(table_hbm.at[:, pl.ds(s, _TW)],
                                 slab_v.at[region * _K + j], sems[region])

        def drain(region):
            for j in range(_K):
                pltpu.make_async_copy(table_hbm.at[:, pl.ds(0, _TW)],
                                      slab_v.at[region * _K + j],
                                      sems[region]).wait()

        def extract(bt, region):
            base = bt * _K
            for j in range(_K):
                v = id_s[base + j]
                sub = jnp.broadcast_to(v & (_TW - 1), (_L,))
                tcol = jnp.broadcast_to(base + j, (_L,))
                for hg in range(_HIDDEN // _L):
                    vals = plsc.load_gather(
                        slab_v,
                        [jnp.broadcast_to(region * _K + j, (_L,)),
                         hvecs[hg], sub])
                    plsc.store_scatter(out_v, [hvecs[hg], tcol], vals)

        spill_ids(0)
        fire(0, 0)
        fire(1, 1)
        fire(2, 2)
        for g in range(1, _NG):
            spill_ids(g)

        def triple_body(i, _):
            bt = 3 * i
            for r in range(3):
                drain(r)
                extract(bt + r, r)
                fire(bt + r + 3, r)
            return _

        lax.fori_loop(0, _NB // 3 - 1, triple_body, 0)
        drain(0)
        extract(_NB - 4, 0)
        fire(_NB - 1, 0)
        drain(1)
        extract(_NB - 3, 1)
        drain(2)
        extract(_NB - 2, 2)
        drain(0)
        extract(_NB - 1, 0)

        pltpu.sync_copy(out_v, out_hbm.at[b, :, pl.ds(l0, _BPW)])

    return gather_kernel


def kernel(input_ids, embedding_weight):
    idx = input_ids.T.reshape(_NW, _BPW).astype(jnp.int32)
    table_t = embedding_weight.T
    out_t = _make_gather()(idx, table_t)
    return jnp.transpose(out_t, (2, 0, 1))
